# baseline (speedup 1.0000x reference)
import functools

import jax
import jax.numpy as jnp
from jax.experimental import pallas as pl

N = 4096
DS = 64
DI = 256
DV = 64
KA = 1024
NH = 4
HD = 16
FF = 256
CH = 512
NCH = N // CH


def _topk_mask(gs):
    gs = gs + 0.0
    i = jax.lax.bitcast_convert_type(gs, jnp.int32)
    key = jnp.where(i < 0, i ^ jnp.int32(0x7FFFFFFF), i)

    cnt_nonneg = jnp.sum((key >= 0).astype(jnp.int32))
    lo = jnp.where(cnt_nonneg >= KA, jnp.int32(0), jnp.int32(-2147483648))
    hi = jnp.where(cnt_nonneg >= KA, jnp.int32(2147483647), jnp.int32(-1))

    def body(_, lohi):
        lo, hi = lohi
        delta = hi - lo
        mid = lo + (delta >> 1) + (delta & 1)
        cnt = jnp.sum((key >= mid).astype(jnp.int32))
        ge = cnt >= KA
        return (jnp.where(ge, mid, lo), jnp.where(ge, hi, mid - 1))

    lo, hi = jax.lax.fori_loop(0, 31, body, (lo, hi))
    tau = lo
    gt = key > tau
    need = KA - jnp.sum(gt.astype(jnp.int32))
    tie = key == tau
    iota = jax.lax.broadcasted_iota(jnp.int32, (1, N), 1)

    def body2(_, lohi):
        lo, hi = lohi
        mid = (lo + hi) >> 1
        cnt = jnp.sum((tie & (iota < mid)).astype(jnp.int32))
        ge = cnt >= need
        return (jnp.where(ge, lo, mid + 1), jnp.where(ge, mid, hi))

    lo2, _ = jax.lax.fori_loop(0, 13, body2,
                               (jnp.int32(0), jnp.int32(N)))
    return gt | (tie & (iota < lo2))


def _step_kernel(*refs):
    t = pl.program_id(0)
    h_sc = refs[-1]
    for b in range(h_sc.shape[0]):
        _batch_body(t, b, *refs)


def _batch_body(t, b, xT_ref, ut_ref, iotak_ref, W_in_ref, normh_ref,
                normx_ref, ffg_h_ref, ffg_x_ref, ff_out_ref, gate_ref,
                vote_ref, wq_ref, wk_ref, wv_ref, attn_out_ref, fs_ref,
                fsh_ref, qrow_ref, onw_ref, out_w_ref, y_ref, h_sc):
    f32 = jnp.float32
    h_b = jnp.where(t == 0, 0.0, h_sc[b])

    gs = jnp.dot(gate_ref[...], h_b, preferred_element_type=f32)
    mask = _topk_mask(gs)
    maskf = mask.astype(f32)

    ut = ut_ref[...]
    iota_k = iotak_ref[...]

    g_cs = []
    off = jnp.float32(0.0)
    h_act = jnp.zeros((DS, KA), f32)
    for c in range(NCH):
        mc = maskf[:, c * CH:(c + 1) * CH]
        rc = jnp.dot(mc, ut, preferred_element_type=f32) + off
        off = off + jnp.sum(mc)
        g_c = ((rc == iota_k) & mask[:, c * CH:(c + 1) * CH]).astype(f32)
        g_cs.append(g_c)
        h_act = h_act + jax.lax.dot_general(
            h_b[:, c * CH:(c + 1) * CH], g_c,
            (((1,), (1,)), ((), ())), preferred_element_type=f32)

    xproj = jnp.dot(W_in_ref[...], xT_ref[0], preferred_element_type=f32)
    x_col = xproj[:, b:b + 1]
    ssq_x = jnp.sum(x_col * x_col)
    ssq_h = jnp.sum(h_act * h_act, axis=0, keepdims=True)
    rr = jax.lax.rsqrt((ssq_h + ssq_x) / (DS + DS + DV) + 1e-6)
    zh = h_act * rr * normh_ref[...]
    ab = (jnp.dot(ffg_h_ref[...], zh, preferred_element_type=f32)
          + jnp.dot(ffg_x_ref[...], x_col * normx_ref[...],
                    preferred_element_type=f32) * rr)
    a_ = ab[:FF]
    b_ = ab[FF:]
    inner = a_ * (b_ * jax.nn.sigmoid(b_))
    h_hat = h_act + jnp.dot(ff_out_ref[...], inner,
                            preferred_element_type=f32)

    v = jnp.dot(vote_ref[...], h_hat, preferred_element_type=f32)
    q = jnp.dot(wq_ref[...], v, preferred_element_type=f32) * 0.25
    k = jnp.dot(wk_ref[...], v, preferred_element_type=f32)
    vv = jnp.dot(wv_ref[...], v, preferred_element_type=f32)
    mt_parts = []
    for hh in range(NH):
        kh = k[HD * hh:HD * (hh + 1)]
        qh = q[HD * hh:HD * (hh + 1)]
        vh = vv[HD * hh:HD * (hh + 1)]
        s_h = jax.lax.dot_general(kh, qh, (((0,), (0,)), ((), ())),
                                  preferred_element_type=f32)
        mx = jnp.max(s_h, axis=0, keepdims=True)
        e = jnp.exp(s_h - mx)
        num = jnp.dot(vh, e, preferred_element_type=f32)
        mt_parts.append(num / jnp.sum(e, axis=0, keepdims=True))
    mt = jnp.concatenate(mt_parts, axis=0)
    m_t = jnp.dot(attn_out_ref[...], mt, preferred_element_type=f32)
    v_mean = jnp.mean(vv, axis=1, keepdims=True)
    m_uni = jnp.dot(attn_out_ref[...], v_mean, preferred_element_type=f32)

    scale = 1.0 + jnp.tanh(jnp.dot(fs_ref[...], m_t,
                                   preferred_element_type=f32))
    shift = jnp.dot(fsh_ref[...], m_t, preferred_element_type=f32)
    h_act_new = h_hat * scale + shift
    su = 1.0 + jnp.tanh(jnp.dot(fs_ref[...], m_uni,
                                preferred_element_type=f32))
    sh_u = jnp.dot(fsh_ref[...], m_uni, preferred_element_type=f32)
    h_dense = h_b * su + sh_u

    parts = []
    for c in range(NCH):
        mc_b = mask[:, c * CH:(c + 1) * CH]
        scat = jnp.dot(h_act_new, g_cs[c], preferred_element_type=f32)
        parts.append(jnp.where(mc_b, scat,
                               h_dense[:, c * CH:(c + 1) * CH]))
    h_new = jnp.concatenate(parts, axis=1)
    h_sc[b] = h_new

    al = jnp.dot(qrow_ref[...], h_new, preferred_element_type=f32) * 0.125
    mx = jnp.max(al, axis=1, keepdims=True)
    e = jnp.exp(al - mx)
    w = e / jnp.sum(e, axis=1, keepdims=True)
    pooled = jnp.sum(h_new * w, axis=1, keepdims=True)
    r = jax.lax.rsqrt(jnp.sum(pooled * pooled) / DS + 1e-6)
    pn = pooled * r * onw_ref[...]
    y_row = jax.lax.dot_general(pn, out_w_ref[...],
                                (((0,), (1,)), ((), ())),
                                preferred_element_type=f32)
    y_ref[0, b] = y_row


@functools.partial(jax.jit, static_argnames=())
def _run(x, W_in, norm_w, ff_g_w, ff_out_w, gate_w, vote_w, attn_in_w,
         attn_out_w, film_scale_w, film_shift_w, query, out_norm_w, out_w):
    B, T, _ = x.shape

    normh = norm_w[:DS].reshape(DS, 1)
    normx = norm_w[DS:2 * DS].reshape(DS, 1)
    ffg_h = ff_g_w[:, :DS]
    ffg_x = ff_g_w[:, DS:2 * DS]
    wq = attn_in_w[:DV]
    wk = attn_in_w[DV:2 * DV]
    wv = attn_in_w[2 * DV:]
    qrow = query.reshape(1, DS)
    onw = out_norm_w.reshape(DS, 1)

    ut = (jnp.arange(CH)[:, None] <= jnp.arange(CH)[None, :]
          ).astype(jnp.float32)
    iota_k = (jnp.arange(KA, dtype=jnp.float32) + 1.0)[:, None]
    iota_k = jnp.broadcast_to(iota_k, (KA, CH))
    xT3 = jnp.transpose(x, (1, 2, 0))

    full = lambda arr: pl.BlockSpec(arr.shape,
                                    lambda t: (0,) * arr.ndim)
    weights = (ut, iota_k, W_in, normh, normx, ffg_h, ffg_x, ff_out_w,
               gate_w, vote_w, wq, wk, wv, attn_out_w, film_scale_w,
               film_shift_w, qrow, onw, out_w)

    from jax.experimental.pallas import tpu as pltpu
    y = pl.pallas_call(
        _step_kernel,
        grid=(T,),
        in_specs=[pl.BlockSpec((1, DI, B), lambda t: (t, 0, 0))]
                 + [full(wgt) for wgt in weights],
        out_specs=[pl.BlockSpec((1, B, 1, DI), lambda t: (t, 0, 0, 0))],
        out_shape=[jax.ShapeDtypeStruct((T, B, 1, DI), jnp.float32)],
        scratch_shapes=[pltpu.VMEM((B, DS, N), jnp.float32)],
    )(xT3, *weights)[0]
    return jnp.transpose(y[:, :, 0], (1, 0, 2))


def kernel(x, W_in, norm_w, ff_g_w, ff_out_w, gate_w, vote_w, attn_in_w,
           attn_out_w, film_scale_w, film_shift_w, query, out_norm_w, out_w):
    return _run(x, W_in, norm_w, ff_g_w, ff_out_w, gate_w, vote_w,
                attn_in_w, attn_out_w, film_scale_w, film_shift_w, query,
                out_norm_w, out_w)


# speedup vs baseline: 1.2079x; 1.2079x over previous
import functools

import jax
import jax.numpy as jnp
from jax.experimental import pallas as pl

N = 4096
DS = 64
DI = 256
DV = 64
KA = 1024
NH = 4
HD = 16
FF = 256
CH = 512
NCH = N // CH


def _topk_mask(gs):
    gs = gs + 0.0
    i = jax.lax.bitcast_convert_type(gs, jnp.int32)
    key = jnp.where(i < 0, i ^ jnp.int32(0x7FFFFFFF), i)

    cnt_nonneg = jnp.sum((key >= 0).astype(jnp.int32))
    lo = jnp.where(cnt_nonneg >= KA, jnp.int32(0), jnp.int32(-2147483648))
    hi = jnp.where(cnt_nonneg >= KA, jnp.int32(2147483647), jnp.int32(-1))

    def body(_, lohi):
        lo, hi = lohi
        delta = hi - lo
        mid = lo + (delta >> 1) + (delta & 1)
        cnt = jnp.sum((key >= mid).astype(jnp.int32))
        ge = cnt >= KA
        return (jnp.where(ge, mid, lo), jnp.where(ge, hi, mid - 1))

    lo, hi = jax.lax.fori_loop(0, 31, body, (lo, hi))
    tau = lo
    gt = key > tau
    need = KA - jnp.sum(gt.astype(jnp.int32))
    tie = key == tau
    iota = jax.lax.broadcasted_iota(jnp.int32, (1, N), 1)

    def body2(_, lohi):
        lo, hi = lohi
        mid = (lo + hi) >> 1
        cnt = jnp.sum((tie & (iota < mid)).astype(jnp.int32))
        ge = cnt >= need
        return (jnp.where(ge, lo, mid + 1), jnp.where(ge, mid, hi))

    lo2, _ = jax.lax.fori_loop(0, 13, body2,
                               (jnp.int32(0), jnp.int32(N)))
    return gt | (tie & (iota < lo2))


def _step_kernel(xT_ref, ut_ref, iotak_ref, W_in_ref, normh_ref,
                 normx_ref, ffg_h_ref, ffg_x_ref, ff_out_ref, gate_ref,
                 vote_ref, wq_ref, wk_ref, wv_ref, attn_out_ref, fs_ref,
                 fsh_ref, qrow_ref, onw_ref, out_w_ref, y_ref, h_sc):
    t = pl.program_id(0)
    b = pl.program_id(1)
    f32 = jnp.float32
    h_b = jnp.where(t == 0, 0.0, h_sc[b])

    gs = jnp.dot(gate_ref[...], h_b, preferred_element_type=f32)
    mask = _topk_mask(gs)
    maskf = mask.astype(f32)

    ut = ut_ref[...]
    iota_k = iotak_ref[...]

    g_cs = []
    off = jnp.float32(0.0)
    h_act = jnp.zeros((DS, KA), f32)
    for c in range(NCH):
        mc = maskf[:, c * CH:(c + 1) * CH]
        rc = (jnp.dot(mc, ut, preferred_element_type=f32) + off) * mc
        off = off + jnp.sum(mc)
        g_c = (rc == iota_k).astype(f32)
        g_cs.append(g_c)
        h_act = h_act + jax.lax.dot_general(
            h_b[:, c * CH:(c + 1) * CH], g_c,
            (((1,), (1,)), ((), ())), preferred_element_type=f32)

    xproj = jnp.dot(W_in_ref[...], xT_ref[0], preferred_element_type=f32)
    nb = xT_ref.shape[2]
    sel = (jax.lax.broadcasted_iota(jnp.int32, (nb, 1), 0) == b).astype(f32)
    x_col = jnp.dot(xproj, sel, preferred_element_type=f32)
    ssq_x = jnp.sum(x_col * x_col)
    ssq_h = jnp.sum(h_act * h_act, axis=0, keepdims=True)
    rr = jax.lax.rsqrt((ssq_h + ssq_x) / (DS + DS + DV) + 1e-6)
    zh = h_act * rr * normh_ref[...]
    ab = (jnp.dot(ffg_h_ref[...], zh, preferred_element_type=f32)
          + jnp.dot(ffg_x_ref[...], x_col * normx_ref[...],
                    preferred_element_type=f32) * rr)
    a_ = ab[:FF]
    b_ = ab[FF:]
    inner = a_ * (b_ * jax.nn.sigmoid(b_))
    h_hat = h_act + jnp.dot(ff_out_ref[...], inner,
                            preferred_element_type=f32)

    v = jnp.dot(vote_ref[...], h_hat, preferred_element_type=f32)
    q = jnp.dot(wq_ref[...], v, preferred_element_type=f32) * 0.25
    k = jnp.dot(wk_ref[...], v, preferred_element_type=f32)
    vv = jnp.dot(wv_ref[...], v, preferred_element_type=f32)
    mt_parts = []
    for hh in range(NH):
        kh = k[HD * hh:HD * (hh + 1)]
        qh = q[HD * hh:HD * (hh + 1)]
        vh = vv[HD * hh:HD * (hh + 1)]
        s_h = jax.lax.dot_general(kh, qh, (((0,), (0,)), ((), ())),
                                  preferred_element_type=f32)
        mx = jnp.max(s_h, axis=0, keepdims=True)
        e = jnp.exp(s_h - mx)
        num = jnp.dot(vh, e, preferred_element_type=f32)
        mt_parts.append(num / jnp.sum(e, axis=0, keepdims=True))
    mt = jnp.concatenate(mt_parts, axis=0)
    m_t = jnp.dot(attn_out_ref[...], mt, preferred_element_type=f32)
    v_mean = jnp.mean(vv, axis=1, keepdims=True)
    m_uni = jnp.dot(attn_out_ref[...], v_mean, preferred_element_type=f32)

    scale = 1.0 + jnp.tanh(jnp.dot(fs_ref[...], m_t,
                                   preferred_element_type=f32))
    shift = jnp.dot(fsh_ref[...], m_t, preferred_element_type=f32)
    h_act_new = h_hat * scale + shift
    su = 1.0 + jnp.tanh(jnp.dot(fs_ref[...], m_uni,
                                preferred_element_type=f32))
    sh_u = jnp.dot(fsh_ref[...], m_uni, preferred_element_type=f32)
    h_dense = h_b * su + sh_u

    parts = []
    for c in range(NCH):
        mc_b = mask[:, c * CH:(c + 1) * CH]
        scat = jnp.dot(h_act_new, g_cs[c], preferred_element_type=f32)
        parts.append(jnp.where(mc_b, scat,
                               h_dense[:, c * CH:(c + 1) * CH]))
    h_new = jnp.concatenate(parts, axis=1)
    h_sc[b] = h_new

    al = jnp.dot(qrow_ref[...], h_new, preferred_element_type=f32) * 0.125
    mx = jnp.max(al, axis=1, keepdims=True)
    e = jnp.exp(al - mx)
    w = e / jnp.sum(e, axis=1, keepdims=True)
    pooled = jnp.sum(h_new * w, axis=1, keepdims=True)
    r = jax.lax.rsqrt(jnp.sum(pooled * pooled) / DS + 1e-6)
    pn = pooled * r * onw_ref[...]
    y_row = jax.lax.dot_general(pn, out_w_ref[...],
                                (((0,), (1,)), ((), ())),
                                preferred_element_type=f32)
    y_ref[0, 0] = y_row


@functools.partial(jax.jit, static_argnames=())
def _run(x, W_in, norm_w, ff_g_w, ff_out_w, gate_w, vote_w, attn_in_w,
         attn_out_w, film_scale_w, film_shift_w, query, out_norm_w, out_w):
    B, T, _ = x.shape

    normh = norm_w[:DS].reshape(DS, 1)
    normx = norm_w[DS:2 * DS].reshape(DS, 1)
    ffg_h = ff_g_w[:, :DS]
    ffg_x = ff_g_w[:, DS:2 * DS]
    wq = attn_in_w[:DV]
    wk = attn_in_w[DV:2 * DV]
    wv = attn_in_w[2 * DV:]
    qrow = query.reshape(1, DS)
    onw = out_norm_w.reshape(DS, 1)

    ut = (jnp.arange(CH)[:, None] <= jnp.arange(CH)[None, :]
          ).astype(jnp.float32)
    iota_k = (jnp.arange(KA, dtype=jnp.float32) + 1.0)[:, None]
    iota_k = jnp.broadcast_to(iota_k, (KA, CH))
    xT3 = jnp.transpose(x, (1, 2, 0))

    full = lambda arr: pl.BlockSpec(arr.shape,
                                    lambda t, b: (0,) * arr.ndim)
    weights = (ut, iota_k, W_in, normh, normx, ffg_h, ffg_x, ff_out_w,
               gate_w, vote_w, wq, wk, wv, attn_out_w, film_scale_w,
               film_shift_w, qrow, onw, out_w)

    from jax.experimental.pallas import tpu as pltpu
    y = pl.pallas_call(
        _step_kernel,
        grid=(T, B),
        in_specs=[pl.BlockSpec((1, DI, B), lambda t, b: (t, 0, 0))]
                 + [full(wgt) for wgt in weights],
        out_specs=[pl.BlockSpec((1, 1, 1, DI), lambda t, b: (t, b, 0, 0))],
        out_shape=[jax.ShapeDtypeStruct((T, B, 1, DI), jnp.float32)],
        scratch_shapes=[pltpu.VMEM((B, DS, N), jnp.float32)],
    )(xT3, *weights)[0]
    return jnp.transpose(y[:, :, 0], (1, 0, 2))


def kernel(x, W_in, norm_w, ff_g_w, ff_out_w, gate_w, vote_w, attn_in_w,
           attn_out_w, film_scale_w, film_shift_w, query, out_norm_w, out_w):
    return _run(x, W_in, norm_w, ff_g_w, ff_out_w, gate_w, vote_w,
                attn_in_w, attn_out_w, film_scale_w, film_shift_w, query,
                out_norm_w, out_w)
